# Initial kernel scaffold; baseline (speedup 1.0000x reference)
import jax
import jax.numpy as jnp
from jax.experimental import pallas as pl


def kernel(x, edge_index, edge_attr, batch, conv_w0, conv_b0, lin_w0, lin_b0, bn_g0, bn_b0, conv_w1, conv_b1, lin_w1, lin_b1, bn_g1, bn_b1, conv_w2, conv_b2, lin_w2, lin_b2, bn_g2, bn_b2):
    raise NotImplementedError("write your pallas kernel here")


# speedup vs baseline: 3.7858x; 3.7858x over previous
import functools

import jax
import jax.numpy as jnp
from jax import lax
from jax.experimental import pallas as pl
from jax.experimental.pallas import tpu as pltpu
from jax.experimental.pallas import tpu_sc as plsc

N = 10000
E = 160000
G = 64
DIM = 256
NE = 3
HALF = 128
NSUB = 16
EPS16 = E // NSUB
K = 128
EPW = 10240
CH = EPW // K
NH = N // 2
ACC = 5120
SRPS = ACC // NSUB
CSLAB = 200
BN = 1000
NB = N // BN
EPS = 1e-5


def _prep_body(src_ref, attr_ref, dst_ref, g_ref, da_ref, db_ref):
    g_ref[...] = attr_ref[...] * N + src_ref[...]
    dst = dst_ref[...]
    da_ref[...] = jnp.where(dst < NH, dst, NH)
    db_ref[...] = jnp.where(dst >= NH, dst - NH, NH)


def _prep_call(src2, attr2, dst2):
    return pl.pallas_call(
        _prep_body,
        out_shape=[jax.ShapeDtypeStruct((NSUB, EPW), jnp.int32)] * 3,
    )(src2, attr2, dst2)


def _ymat_body(h_ref, w_ref, y0_ref, y1_ref):
    w = w_ref[0]
    out = lax.dot_general(h_ref[...], w, (((1,), (1,)), ((), ())),
                          preferred_element_type=jnp.float32)
    y0_ref[...] = out[:, :HALF]
    y1_ref[...] = out[:, HALF:]


def _ymat_call(h, cw):
    din = h.shape[1]
    return pl.pallas_call(
        _ymat_body,
        grid=(NE, NB),
        in_specs=[
            pl.BlockSpec((BN, din), lambda et, i: (i, 0)),
            pl.BlockSpec((1, DIM, din), lambda et, i: (et, 0, 0)),
        ],
        out_specs=[
            pl.BlockSpec((BN, HALF), lambda et, i: (et * NB + i, 0)),
            pl.BlockSpec((BN, HALF), lambda et, i: (et * NB + i, 0)),
        ],
        out_shape=[jax.ShapeDtypeStruct((NE * N, HALF), jnp.float32)] * 2,
    )(h, cw)


def _layer_body(h_ref, a0_ref, a1_ref, lw_ref, lb_ref, cb_ref, b3_ref,
                hr_ref, stats_ref, seg_ref, cnt_ref):
    i = pl.program_id(0)
    out = lax.dot_general(h_ref[...], lw_ref[...], (((1,), (1,)), ((), ())),
                          preferred_element_type=jnp.float32)
    bias = lb_ref[0, :] + cb_ref[0, :] + cb_ref[1, :] + cb_ref[2, :]
    out = out + bias[None, :]
    out = out + jnp.concatenate([a0_ref[...], a1_ref[...]], axis=1)
    hr = jnp.maximum(out, 0.0)
    hr_ref[...] = hr

    @pl.when(i == 0)
    def _():
        stats_ref[...] = jnp.zeros_like(stats_ref)
        seg_ref[...] = jnp.zeros_like(seg_ref)
        cnt_ref[...] = jnp.zeros_like(cnt_ref)

    stats_ref[0:1, :] += jnp.sum(hr, axis=0)[None, :]
    stats_ref[1:2, :] += jnp.sum(hr * hr, axis=0)[None, :]
    b = b3_ref[0]
    oh = (lax.broadcasted_iota(jnp.int32, (G, BN), 0) == b).astype(jnp.float32)
    seg_ref[...] += lax.dot_general(oh, hr, (((1,), (0,)), ((), ())),
                                    preferred_element_type=jnp.float32)
    cnt_ref[...] += jnp.broadcast_to(jnp.sum(oh, axis=1)[:, None], (G, HALF))


def _layer_call(h, a0, a1, lw, lb2, cb, batch3):
    din = h.shape[1]
    return pl.pallas_call(
        _layer_body,
        grid=(NB,),
        in_specs=[
            pl.BlockSpec((BN, din), lambda i: (i, 0)),
            pl.BlockSpec((BN, HALF), lambda i: (i, 0)),
            pl.BlockSpec((BN, HALF), lambda i: (i, 0)),
            pl.BlockSpec((DIM, din), lambda i: (0, 0)),
            pl.BlockSpec((1, DIM), lambda i: (0, 0)),
            pl.BlockSpec((NE, DIM), lambda i: (0, 0)),
            pl.BlockSpec((1, 1, BN), lambda i: (i, 0, 0)),
        ],
        out_specs=[
            pl.BlockSpec((BN, DIM), lambda i: (i, 0)),
            pl.BlockSpec((8, DIM), lambda i: (0, 0)),
            pl.BlockSpec((G, DIM), lambda i: (0, 0)),
            pl.BlockSpec((G, HALF), lambda i: (0, 0)),
        ],
        out_shape=[
            jax.ShapeDtypeStruct((N, DIM), jnp.float32),
            jax.ShapeDtypeStruct((8, DIM), jnp.float32),
            jax.ShapeDtypeStruct((G, DIM), jnp.float32),
            jax.ShapeDtypeStruct((G, HALF), jnp.float32),
        ],
    )(h, a0, a1, lw, lb2, cb, batch3)


def _norm_body(h_ref, stats_ref, g_ref, b_ref, o_ref):
    mean = stats_ref[0:1, :] * (1.0 / N)
    var = stats_ref[1:2, :] * (1.0 / N) - mean * mean
    sc = lax.rsqrt(var + EPS) * g_ref[...]
    sh = b_ref[...] - mean * sc
    o_ref[...] = h_ref[...] * sc + sh


def _norm_call(hr, stats, bg2, bb2):
    return pl.pallas_call(
        _norm_body,
        grid=(NB,),
        in_specs=[
            pl.BlockSpec((BN, DIM), lambda i: (i, 0)),
            pl.BlockSpec((8, DIM), lambda i: (0, 0)),
            pl.BlockSpec((1, DIM), lambda i: (0, 0)),
            pl.BlockSpec((1, DIM), lambda i: (0, 0)),
        ],
        out_specs=pl.BlockSpec((BN, DIM), lambda i: (i, 0)),
        out_shape=jax.ShapeDtypeStruct((N, DIM), jnp.float32),
    )(hr, stats, bg2, bb2)


def _pool_body(seg_ref, cnt_ref, stats_ref, g_ref, b_ref, o_ref):
    mean = stats_ref[0:1, :] * (1.0 / N)
    var = stats_ref[1:2, :] * (1.0 / N) - mean * mean
    inv = lax.rsqrt(var + EPS)
    cnt = cnt_ref[:, 0:1]
    pooled = (seg_ref[...] - cnt * mean) * (inv * g_ref[...]) + cnt * b_ref[...]
    o_ref[...] = pooled * lax.rsqrt(cnt)


def _pool_call(seg, cnt, stats, bg2, bb2):
    return pl.pallas_call(
        _pool_body,
        out_shape=jax.ShapeDtypeStruct((G, DIM), jnp.float32),
    )(seg, cnt, stats, bg2, bb2)


@functools.lru_cache(maxsize=None)
def _sc_scatter_kernel():
    mesh = plsc.VectorSubcoreMesh(core_axis_name="c", subcore_axis_name="s")

    @functools.partial(
        pl.kernel,
        out_type=(jax.ShapeDtypeStruct((N, HALF), jnp.float32),
                  jax.ShapeDtypeStruct((N, HALF), jnp.float32)),
        mesh=mesh,
        scratch_types=[
            pltpu.VMEM((CH, K), jnp.int32),
            pltpu.VMEM((CH, K), jnp.int32),
            pltpu.VMEM((CH, K), jnp.int32),
            pltpu.VMEM((K, HALF), jnp.float32),
            pltpu.VMEM((K, HALF), jnp.float32),
            pltpu.VMEM((K, HALF), jnp.float32),
            pltpu.VMEM_SHARED((ACC, HALF), jnp.float32),
            pltpu.SemaphoreType.DMA,
            pltpu.SemaphoreType.DMA,
        ],
    )
    def _sc_scatter(y0_hbm, y1_hbm, gidx_hbm, dsta_hbm, dstb_hbm,
                    agg0_hbm, agg1_hbm,
                    gidx_t, dsta_t, dstb_t, rows0, rows1, zbuf, acc,
                    sem0, sem1):
        c = lax.axis_index("c")
        s = lax.axis_index("s")
        pltpu.sync_copy(gidx_hbm.at[s], gidx_t)
        pltpu.sync_copy(dsta_hbm.at[s], dsta_t)
        pltpu.sync_copy(dstb_hbm.at[s], dstb_t)

        @pl.loop(0, K)
        def _(r):
            @pl.loop(0, HALF, step=16)
            def _(f):
                zbuf[r, pl.ds(f, 16)] = jnp.zeros((16,), jnp.float32)

        zbase = s * SRPS

        def do_pass(y_hbm, dst_t, agg_hbm, out_base):
            for k in range(SRPS // K):
                pltpu.sync_copy(zbuf, acc.at[pl.ds(zbase + k * K, K)])
            pltpu.sync_copy(zbuf.at[pl.ds(0, SRPS % K)],
                            acc.at[pl.ds(zbase + (SRPS // K) * K, SRPS % K)])
            plsc.subcore_barrier()

            bufs = (rows0, rows1)
            sems = (sem0, sem1)
            for b in range(2):
                pltpu.make_async_copy(y_hbm.at[gidx_t.at[b]], bufs[b],
                                      sems[b]).start()

            @pl.loop(0, CH, step=2)
            def _(j):
                for b in range(2):
                    jj = j + b
                    pltpu.make_async_copy(y_hbm.at[gidx_t.at[jj]], bufs[b],
                                          sems[b]).wait()
                    pltpu.sync_copy(bufs[b], acc.at[dst_t.at[jj]], add=True)

                    @pl.when(jj + 2 < CH)
                    def _():
                        pltpu.make_async_copy(y_hbm.at[gidx_t.at[jj + 2]],
                                              bufs[b], sems[b]).start()

            plsc.subcore_barrier()

            pltpu.sync_copy(
                acc.at[pl.ds(s * CSLAB, CSLAB)],
                agg_hbm.at[pl.ds(out_base + s * CSLAB, CSLAB)])

            @pl.when(s < 9)
            def _():
                pltpu.sync_copy(
                    acc.at[pl.ds((s + 16) * CSLAB, CSLAB)],
                    agg_hbm.at[pl.ds(out_base + (s + 16) * CSLAB, CSLAB)])
            plsc.subcore_barrier()

        def do_core(y_hbm, agg_hbm):
            do_pass(y_hbm, dsta_t, agg_hbm, 0)
            do_pass(y_hbm, dstb_t, agg_hbm, NH)

        @pl.when(c == 0)
        def _():
            do_core(y0_hbm, agg0_hbm)

        @pl.when(c == 1)
        def _():
            do_core(y1_hbm, agg1_hbm)

    return _sc_scatter


def _sc_apply(y0, y1, gidx3, dsta3, dstb3):
    return _sc_scatter_kernel()(y0, y1, gidx3, dsta3, dstb3)


def kernel(x, edge_index, edge_attr, batch,
           conv_w0, conv_b0, lin_w0, lin_b0, bn_g0, bn_b0,
           conv_w1, conv_b1, lin_w1, lin_b1, bn_g1, bn_b1,
           conv_w2, conv_b2, lin_w2, lin_b2, bn_g2, bn_b2):
    pad = EPW - EPS16
    src2 = jnp.pad(edge_index[0].reshape(NSUB, EPS16), ((0, 0), (0, pad)))
    attr2 = jnp.pad(edge_attr.reshape(NSUB, EPS16), ((0, 0), (0, pad)))
    dst2 = jnp.pad(edge_index[1].reshape(NSUB, EPS16), ((0, 0), (0, pad)),
                   constant_values=N)
    gidx, dsta, dstb = _prep_call(src2, attr2, dst2)
    gidx3 = gidx.reshape(NSUB, CH, K)
    dsta3 = dsta.reshape(NSUB, CH, K)
    dstb3 = dstb.reshape(NSUB, CH, K)
    batch3 = batch.reshape(NB, 1, BN)

    params = [
        (conv_w0, conv_b0, lin_w0, lin_b0, bn_g0, bn_b0),
        (conv_w1, conv_b1, lin_w1, lin_b1, bn_g1, bn_b1),
        (conv_w2, conv_b2, lin_w2, lin_b2, bn_g2, bn_b2),
    ]
    h = x
    pooled = []
    for l, (cw, cb, lw, lb, bg, bb) in enumerate(params):
        y0, y1 = _ymat_call(h, cw)
        a0, a1 = _sc_apply(y0, y1, gidx3, dsta3, dstb3)
        hr, stats, seg, cnt = _layer_call(h, a0, a1, lw, lb.reshape(1, DIM),
                                          cb, batch3)
        bg2, bb2 = bg.reshape(1, DIM), bb.reshape(1, DIM)
        if l < 2:
            h = _norm_call(hr, stats, bg2, bb2)
        pooled.append(_pool_call(seg, cnt, stats, bg2, bb2))
    return jnp.concatenate(pooled, axis=1)


# speedup vs baseline: 4.2119x; 1.1126x over previous
import functools

import jax
import jax.numpy as jnp
from jax import lax
from jax.experimental import pallas as pl
from jax.experimental.pallas import tpu as pltpu
from jax.experimental.pallas import tpu_sc as plsc

N = 10000
E = 160000
G = 64
DIM = 256
NE = 3
HALF = 128
NSUB = 16
EPS16 = E // NSUB
K = 128
EPW = 10240
KR = 1
KROWS = KR * K
CH = EPW // KROWS
NH = N // 2
ACC = 5120
SRPS = ACC // NSUB
CSLAB = 200
BN = 1000
NB = N // BN
EPS = 1e-5


def _prep_body(src_ref, attr_ref, dst_ref, g_ref, da_ref, db_ref):
    g_ref[...] = attr_ref[...] * N + src_ref[...]
    dst = dst_ref[...]
    rowi = lax.broadcasted_iota(jnp.int32, (NSUB, EPW), 0)
    da_ref[...] = jnp.where(dst < NH, dst, NH + rowi)
    db_ref[...] = jnp.where(dst >= NH, dst - NH, NH + rowi)


def _prep_call(src2, attr2, dst2):
    return pl.pallas_call(
        _prep_body,
        out_shape=[jax.ShapeDtypeStruct((NSUB, EPW), jnp.int32)] * 3,
    )(src2, attr2, dst2)


def _ymat_body(h_ref, w_ref, y0_ref, y1_ref):
    w = w_ref[0]
    out = lax.dot_general(h_ref[...], w, (((1,), (1,)), ((), ())),
                          preferred_element_type=jnp.float32)
    y0_ref[...] = out[:, :HALF]
    y1_ref[...] = out[:, HALF:]


def _ymat_call(h, cw):
    din = h.shape[1]
    return pl.pallas_call(
        _ymat_body,
        grid=(NE, NB),
        in_specs=[
            pl.BlockSpec((BN, din), lambda et, i: (i, 0)),
            pl.BlockSpec((1, DIM, din), lambda et, i: (et, 0, 0)),
        ],
        out_specs=[
            pl.BlockSpec((BN, HALF), lambda et, i: (et * NB + i, 0)),
            pl.BlockSpec((BN, HALF), lambda et, i: (et * NB + i, 0)),
        ],
        out_shape=[jax.ShapeDtypeStruct((NE * N, HALF), jnp.float32)] * 2,
    )(h, cw)


def _layer_body(h_ref, a0_ref, a1_ref, lw_ref, lb_ref, cb_ref, b3_ref,
                hr_ref, stats_ref, seg_ref, cnt_ref):
    i = pl.program_id(0)
    out = lax.dot_general(h_ref[...], lw_ref[...], (((1,), (1,)), ((), ())),
                          preferred_element_type=jnp.float32)
    bias = lb_ref[0, :] + cb_ref[0, :] + cb_ref[1, :] + cb_ref[2, :]
    out = out + bias[None, :]
    out = out + jnp.concatenate([a0_ref[...], a1_ref[...]], axis=1)
    hr = jnp.maximum(out, 0.0)
    hr_ref[...] = hr

    @pl.when(i == 0)
    def _():
        stats_ref[...] = jnp.zeros_like(stats_ref)
        seg_ref[...] = jnp.zeros_like(seg_ref)
        cnt_ref[...] = jnp.zeros_like(cnt_ref)

    stats_ref[0:1, :] += jnp.sum(hr, axis=0)[None, :]
    stats_ref[1:2, :] += jnp.sum(hr * hr, axis=0)[None, :]
    b = b3_ref[0]
    oh = (lax.broadcasted_iota(jnp.int32, (G, BN), 0) == b).astype(jnp.float32)
    seg_ref[...] += lax.dot_general(oh, hr, (((1,), (0,)), ((), ())),
                                    preferred_element_type=jnp.float32)
    cnt_ref[...] += jnp.broadcast_to(jnp.sum(oh, axis=1)[:, None], (G, HALF))


def _layer_call(h, a0, a1, lw, lb2, cb, batch3):
    din = h.shape[1]
    return pl.pallas_call(
        _layer_body,
        grid=(NB,),
        in_specs=[
            pl.BlockSpec((BN, din), lambda i: (i, 0)),
            pl.BlockSpec((BN, HALF), lambda i: (i, 0)),
            pl.BlockSpec((BN, HALF), lambda i: (i, 0)),
            pl.BlockSpec((DIM, din), lambda i: (0, 0)),
            pl.BlockSpec((1, DIM), lambda i: (0, 0)),
            pl.BlockSpec((NE, DIM), lambda i: (0, 0)),
            pl.BlockSpec((1, 1, BN), lambda i: (i, 0, 0)),
        ],
        out_specs=[
            pl.BlockSpec((BN, DIM), lambda i: (i, 0)),
            pl.BlockSpec((8, DIM), lambda i: (0, 0)),
            pl.BlockSpec((G, DIM), lambda i: (0, 0)),
            pl.BlockSpec((G, HALF), lambda i: (0, 0)),
        ],
        out_shape=[
            jax.ShapeDtypeStruct((N, DIM), jnp.float32),
            jax.ShapeDtypeStruct((8, DIM), jnp.float32),
            jax.ShapeDtypeStruct((G, DIM), jnp.float32),
            jax.ShapeDtypeStruct((G, HALF), jnp.float32),
        ],
    )(h, a0, a1, lw, lb2, cb, batch3)


def _norm_body(h_ref, stats_ref, g_ref, b_ref, o_ref):
    mean = stats_ref[0:1, :] * (1.0 / N)
    var = stats_ref[1:2, :] * (1.0 / N) - mean * mean
    sc = lax.rsqrt(var + EPS) * g_ref[...]
    sh = b_ref[...] - mean * sc
    o_ref[...] = h_ref[...] * sc + sh


def _norm_call(hr, stats, bg2, bb2):
    return pl.pallas_call(
        _norm_body,
        grid=(NB,),
        in_specs=[
            pl.BlockSpec((BN, DIM), lambda i: (i, 0)),
            pl.BlockSpec((8, DIM), lambda i: (0, 0)),
            pl.BlockSpec((1, DIM), lambda i: (0, 0)),
            pl.BlockSpec((1, DIM), lambda i: (0, 0)),
        ],
        out_specs=pl.BlockSpec((BN, DIM), lambda i: (i, 0)),
        out_shape=jax.ShapeDtypeStruct((N, DIM), jnp.float32),
    )(hr, stats, bg2, bb2)


def _pool_body(seg_ref, cnt_ref, stats_ref, g_ref, b_ref, o_ref):
    mean = stats_ref[0:1, :] * (1.0 / N)
    var = stats_ref[1:2, :] * (1.0 / N) - mean * mean
    inv = lax.rsqrt(var + EPS)
    cnt = cnt_ref[:, 0:1]
    pooled = (seg_ref[...] - cnt * mean) * (inv * g_ref[...]) + cnt * b_ref[...]
    o_ref[...] = pooled * lax.rsqrt(cnt)


def _pool_call(seg, cnt, stats, bg2, bb2):
    return pl.pallas_call(
        _pool_body,
        out_shape=jax.ShapeDtypeStruct((G, DIM), jnp.float32),
    )(seg, cnt, stats, bg2, bb2)


@functools.lru_cache(maxsize=None)
def _sc_scatter_kernel():
    mesh = plsc.VectorSubcoreMesh(core_axis_name="c", subcore_axis_name="s")

    @functools.partial(
        pl.kernel,
        out_type=(jax.ShapeDtypeStruct((N, HALF), jnp.float32),
                  jax.ShapeDtypeStruct((N, HALF), jnp.float32)),
        mesh=mesh,
        scratch_types=[
            pltpu.VMEM((CH, KROWS), jnp.int32),
            pltpu.VMEM((CH, KROWS), jnp.int32),
            pltpu.VMEM((CH, KROWS), jnp.int32),
            pltpu.VMEM((KROWS, HALF), jnp.float32),
            pltpu.VMEM((KROWS, HALF), jnp.float32),
            pltpu.VMEM((K, HALF), jnp.float32),
            pltpu.VMEM_SHARED((ACC, HALF), jnp.float32),
            pltpu.SemaphoreType.DMA,
            pltpu.SemaphoreType.DMA,
        ],
    )
    def _sc_scatter(y0_hbm, y1_hbm, gidx_hbm, dsta_hbm, dstb_hbm,
                    agg0_hbm, agg1_hbm,
                    gidx_t, dsta_t, dstb_t, rows0, rows1, zbuf, acc,
                    sem0, sem1):
        c = lax.axis_index("c")
        s = lax.axis_index("s")
        pltpu.sync_copy(gidx_hbm.at[s], gidx_t)
        pltpu.sync_copy(dsta_hbm.at[s], dsta_t)
        pltpu.sync_copy(dstb_hbm.at[s], dstb_t)

        @pl.loop(0, K)
        def _(r):
            @pl.loop(0, HALF, step=16)
            def _(f):
                zbuf[r, pl.ds(f, 16)] = jnp.zeros((16,), jnp.float32)

        zbase = s * SRPS

        def do_pass(y_hbm, dst_t, agg_hbm, out_base):
            for k in range(SRPS // K):
                pltpu.sync_copy(zbuf, acc.at[pl.ds(zbase + k * K, K)])
            pltpu.sync_copy(zbuf.at[pl.ds(0, SRPS % K)],
                            acc.at[pl.ds(zbase + (SRPS // K) * K, SRPS % K)])
            plsc.subcore_barrier()

            bufs = (rows0, rows1)
            sems = (sem0, sem1)
            for b in range(2):
                pltpu.make_async_copy(
                    y_hbm.at[gidx_t.at[b]], bufs[b],
                    sems[b]).start()

            @pl.loop(0, CH, step=2)
            def _(j):
                for b in range(2):
                    jj = j + b
                    pltpu.make_async_copy(
                        y_hbm.at[gidx_t.at[jj]], bufs[b],
                        sems[b]).wait()
                    pltpu.sync_copy(bufs[b],
                                    acc.at[dst_t.at[jj]],
                                    add=True)

                    @pl.when(jj + 2 < CH)
                    def _():
                        pltpu.make_async_copy(
                            y_hbm.at[gidx_t.at[jj + 2]],
                            bufs[b], sems[b]).start()

            plsc.subcore_barrier()

            pltpu.sync_copy(
                acc.at[pl.ds(s * CSLAB, CSLAB)],
                agg_hbm.at[pl.ds(out_base + s * CSLAB, CSLAB)])

            @pl.when(s < 9)
            def _():
                pltpu.sync_copy(
                    acc.at[pl.ds((s + 16) * CSLAB, CSLAB)],
                    agg_hbm.at[pl.ds(out_base + (s + 16) * CSLAB, CSLAB)])
            plsc.subcore_barrier()

        def do_core(y_hbm, agg_hbm):
            do_pass(y_hbm, dsta_t, agg_hbm, 0)
            do_pass(y_hbm, dstb_t, agg_hbm, NH)

        @pl.when(c == 0)
        def _():
            do_core(y0_hbm, agg0_hbm)

        @pl.when(c == 1)
        def _():
            do_core(y1_hbm, agg1_hbm)

    return _sc_scatter


def _sc_apply(y0, y1, gidx3, dsta3, dstb3):
    return _sc_scatter_kernel()(y0, y1, gidx3, dsta3, dstb3)


def kernel(x, edge_index, edge_attr, batch,
           conv_w0, conv_b0, lin_w0, lin_b0, bn_g0, bn_b0,
           conv_w1, conv_b1, lin_w1, lin_b1, bn_g1, bn_b1,
           conv_w2, conv_b2, lin_w2, lin_b2, bn_g2, bn_b2):
    pad = EPW - EPS16
    src2 = jnp.pad(edge_index[0].reshape(NSUB, EPS16), ((0, 0), (0, pad)))
    attr2 = jnp.pad(edge_attr.reshape(NSUB, EPS16), ((0, 0), (0, pad)))
    dst2 = jnp.pad(edge_index[1].reshape(NSUB, EPS16), ((0, 0), (0, pad)),
                   constant_values=N)
    gidx, dsta, dstb = _prep_call(src2, attr2, dst2)
    gidx3 = gidx.reshape(NSUB, CH, KROWS)
    dsta3 = dsta.reshape(NSUB, CH, KROWS)
    dstb3 = dstb.reshape(NSUB, CH, KROWS)
    batch3 = batch.reshape(NB, 1, BN)

    params = [
        (conv_w0, conv_b0, lin_w0, lin_b0, bn_g0, bn_b0),
        (conv_w1, conv_b1, lin_w1, lin_b1, bn_g1, bn_b1),
        (conv_w2, conv_b2, lin_w2, lin_b2, bn_g2, bn_b2),
    ]
    h = x
    pooled = []
    for l, (cw, cb, lw, lb, bg, bb) in enumerate(params):
        y0, y1 = _ymat_call(h, cw)
        a0, a1 = _sc_apply(y0, y1, gidx3, dsta3, dstb3)
        hr, stats, seg, cnt = _layer_call(h, a0, a1, lw, lb.reshape(1, DIM),
                                          cb, batch3)
        bg2, bb2 = bg.reshape(1, DIM), bb.reshape(1, DIM)
        if l < 2:
            h = _norm_call(hr, stats, bg2, bb2)
        pooled.append(_pool_call(seg, cnt, stats, bg2, bb2))
    return jnp.concatenate(pooled, axis=1)


# speedup vs baseline: 7.3157x; 1.7369x over previous
import functools

import jax
import jax.numpy as jnp
from jax import lax
from jax.experimental import pallas as pl
from jax.experimental.pallas import tpu as pltpu
from jax.experimental.pallas import tpu_sc as plsc

N = 10000
E = 160000
G = 64
DIM = 256
NE = 3
HALF = 128
NSUB = 16
EPS16 = E // NSUB
K = 128
EPW = 10240
CH = EPW // K
HCH = CH // 2
ACC = 10240
SRPS = ACC // NSUB
CSLAB = 200
BN = 1000
NB = N // BN
EPS = 1e-5


def _prep_body(src_ref, attr_ref, dst_ref, g_ref, dp_ref):
    g_ref[...] = attr_ref[...] * N + src_ref[...]
    dst = dst_ref[...]
    rowi = lax.broadcasted_iota(jnp.int32, (NSUB, EPW), 0)
    dp_ref[...] = jnp.where(dst < N, dst, N + rowi)


def _prep_call(src2, attr2, dst2):
    return pl.pallas_call(
        _prep_body,
        out_shape=[jax.ShapeDtypeStruct((NSUB, EPW), jnp.int32)] * 2,
    )(src2, attr2, dst2)


def _ymat_body(h_ref, w_ref, y0_ref, y1_ref):
    w = w_ref[0]
    out = lax.dot_general(h_ref[...], w, (((1,), (1,)), ((), ())),
                          preferred_element_type=jnp.float32)
    y0_ref[...] = out[:, :HALF]
    y1_ref[...] = out[:, HALF:]


def _ymat_call(h, cw):
    din = h.shape[1]
    return pl.pallas_call(
        _ymat_body,
        grid=(NE, NB),
        in_specs=[
            pl.BlockSpec((BN, din), lambda et, i: (i, 0)),
            pl.BlockSpec((1, DIM, din), lambda et, i: (et, 0, 0)),
        ],
        out_specs=[
            pl.BlockSpec((BN, HALF), lambda et, i: (et * NB + i, 0)),
            pl.BlockSpec((BN, HALF), lambda et, i: (et * NB + i, 0)),
        ],
        out_shape=[jax.ShapeDtypeStruct((NE * N, HALF), jnp.float32)] * 2,
    )(h, cw)


def _layer_body(h_ref, a0_ref, a1_ref, lw_ref, lb_ref, cb_ref, b3_ref,
                hr_ref, stats_ref, seg_ref, cnt_ref):
    i = pl.program_id(0)
    out = lax.dot_general(h_ref[...], lw_ref[...], (((1,), (1,)), ((), ())),
                          preferred_element_type=jnp.float32)
    bias = lb_ref[0, :] + cb_ref[0, :] + cb_ref[1, :] + cb_ref[2, :]
    out = out + bias[None, :]
    out = out + jnp.concatenate([a0_ref[...], a1_ref[...]], axis=1)
    hr = jnp.maximum(out, 0.0)
    hr_ref[...] = hr

    @pl.when(i == 0)
    def _():
        stats_ref[...] = jnp.zeros_like(stats_ref)
        seg_ref[...] = jnp.zeros_like(seg_ref)
        cnt_ref[...] = jnp.zeros_like(cnt_ref)

    stats_ref[0:1, :] += jnp.sum(hr, axis=0)[None, :]
    stats_ref[1:2, :] += jnp.sum(hr * hr, axis=0)[None, :]
    b = b3_ref[0]
    oh = (lax.broadcasted_iota(jnp.int32, (G, BN), 0) == b).astype(jnp.float32)
    seg_ref[...] += lax.dot_general(oh, hr, (((1,), (0,)), ((), ())),
                                    preferred_element_type=jnp.float32)
    cnt_ref[...] += jnp.broadcast_to(jnp.sum(oh, axis=1)[:, None], (G, HALF))


def _layer_call(h, a0, a1, lw, lb2, cb, batch3):
    din = h.shape[1]
    return pl.pallas_call(
        _layer_body,
        grid=(NB,),
        in_specs=[
            pl.BlockSpec((BN, din), lambda i: (i, 0)),
            pl.BlockSpec((BN, HALF), lambda i: (i, 0)),
            pl.BlockSpec((BN, HALF), lambda i: (i, 0)),
            pl.BlockSpec((DIM, din), lambda i: (0, 0)),
            pl.BlockSpec((1, DIM), lambda i: (0, 0)),
            pl.BlockSpec((NE, DIM), lambda i: (0, 0)),
            pl.BlockSpec((1, 1, BN), lambda i: (i, 0, 0)),
        ],
        out_specs=[
            pl.BlockSpec((BN, DIM), lambda i: (i, 0)),
            pl.BlockSpec((8, DIM), lambda i: (0, 0)),
            pl.BlockSpec((G, DIM), lambda i: (0, 0)),
            pl.BlockSpec((G, HALF), lambda i: (0, 0)),
        ],
        out_shape=[
            jax.ShapeDtypeStruct((N, DIM), jnp.float32),
            jax.ShapeDtypeStruct((8, DIM), jnp.float32),
            jax.ShapeDtypeStruct((G, DIM), jnp.float32),
            jax.ShapeDtypeStruct((G, HALF), jnp.float32),
        ],
    )(h, a0, a1, lw, lb2, cb, batch3)


def _norm_body(h_ref, stats_ref, g_ref, b_ref, o_ref):
    mean = stats_ref[0:1, :] * (1.0 / N)
    var = stats_ref[1:2, :] * (1.0 / N) - mean * mean
    sc = lax.rsqrt(var + EPS) * g_ref[...]
    sh = b_ref[...] - mean * sc
    o_ref[...] = h_ref[...] * sc + sh


def _norm_call(hr, stats, bg2, bb2):
    return pl.pallas_call(
        _norm_body,
        grid=(NB,),
        in_specs=[
            pl.BlockSpec((BN, DIM), lambda i: (i, 0)),
            pl.BlockSpec((8, DIM), lambda i: (0, 0)),
            pl.BlockSpec((1, DIM), lambda i: (0, 0)),
            pl.BlockSpec((1, DIM), lambda i: (0, 0)),
        ],
        out_specs=pl.BlockSpec((BN, DIM), lambda i: (i, 0)),
        out_shape=jax.ShapeDtypeStruct((N, DIM), jnp.float32),
    )(hr, stats, bg2, bb2)


def _pool_body(seg_ref, cnt_ref, stats_ref, g_ref, b_ref, o_ref):
    mean = stats_ref[0:1, :] * (1.0 / N)
    var = stats_ref[1:2, :] * (1.0 / N) - mean * mean
    inv = lax.rsqrt(var + EPS)
    cnt = cnt_ref[:, 0:1]
    pooled = (seg_ref[...] - cnt * mean) * (inv * g_ref[...]) + cnt * b_ref[...]
    o_ref[...] = pooled * lax.rsqrt(cnt)


def _pool_call(seg, cnt, stats, bg2, bb2):
    return pl.pallas_call(
        _pool_body,
        out_shape=jax.ShapeDtypeStruct((G, DIM), jnp.float32),
    )(seg, cnt, stats, bg2, bb2)


@functools.lru_cache(maxsize=None)
def _sc_scatter_kernel():
    mesh = plsc.VectorSubcoreMesh(core_axis_name="c", subcore_axis_name="s")

    @functools.partial(
        pl.kernel,
        out_type=(jax.ShapeDtypeStruct((N, HALF), jnp.float32),
                  jax.ShapeDtypeStruct((N, HALF), jnp.float32)),
        mesh=mesh,
        scratch_types=[
            pltpu.VMEM((CH, K), jnp.int32),
            pltpu.VMEM((HCH, K), jnp.int32),
            pltpu.VMEM((K, HALF), jnp.float32),
            pltpu.VMEM((K, HALF), jnp.float32),
            pltpu.VMEM_SHARED((ACC, HALF), jnp.float32),
            pltpu.SemaphoreType.DMA,
            pltpu.SemaphoreType.DMA,
        ],
    )
    def _sc_scatter(y0_hbm, y1_hbm, gidx_hbm, dst_hbm,
                    agg0_hbm, agg1_hbm,
                    gidx_t, dst_t, rows0, rows1, acc, sem0, sem1):
        c = lax.axis_index("c")
        s = lax.axis_index("s")
        pltpu.sync_copy(gidx_hbm.at[s], gidx_t)

        @pl.loop(0, K)
        def _(r):
            @pl.loop(0, HALF, step=16)
            def _(f):
                rows0[r, pl.ds(f, 16)] = jnp.zeros((16,), jnp.float32)

        zbase = s * SRPS
        for k in range(SRPS // K):
            pltpu.sync_copy(rows0, acc.at[pl.ds(zbase + k * K, K)])
        plsc.subcore_barrier()

        def run_core(y_hbm, agg_hbm):
            bufs = (rows0, rows1)
            sems = (sem0, sem1)
            for half in range(2):
                pltpu.sync_copy(dst_hbm.at[2 * s + half], dst_t)
                for b in range(2):
                    pltpu.make_async_copy(
                        y_hbm.at[gidx_t.at[half * HCH + b]], bufs[b],
                        sems[b]).start()

                @pl.loop(0, HCH, step=2)
                def _(j):
                    for b in range(2):
                        jj = j + b
                        pltpu.make_async_copy(
                            y_hbm.at[gidx_t.at[half * HCH + jj]], bufs[b],
                            sems[b]).wait()
                        pltpu.sync_copy(bufs[b], acc.at[dst_t.at[jj]],
                                        add=True)

                        @pl.when(jj + 2 < HCH)
                        def _():
                            pltpu.make_async_copy(
                                y_hbm.at[gidx_t.at[half * HCH + jj + 2]],
                                bufs[b], sems[b]).start()

            plsc.subcore_barrier()

            for q in range(3):
                slab = s + 16 * q
                pltpu.sync_copy(
                    acc.at[pl.ds(slab * CSLAB, CSLAB)],
                    agg_hbm.at[pl.ds(slab * CSLAB, CSLAB)])

            @pl.when(s < 2)
            def _():
                slab = s + 48
                pltpu.sync_copy(
                    acc.at[pl.ds(slab * CSLAB, CSLAB)],
                    agg_hbm.at[pl.ds(slab * CSLAB, CSLAB)])

        @pl.when(c == 0)
        def _():
            run_core(y0_hbm, agg0_hbm)

        @pl.when(c == 1)
        def _():
            run_core(y1_hbm, agg1_hbm)

    return _sc_scatter


def _sc_apply(y0, y1, gidx3, dst3):
    return _sc_scatter_kernel()(y0, y1, gidx3, dst3)


def kernel(x, edge_index, edge_attr, batch,
           conv_w0, conv_b0, lin_w0, lin_b0, bn_g0, bn_b0,
           conv_w1, conv_b1, lin_w1, lin_b1, bn_g1, bn_b1,
           conv_w2, conv_b2, lin_w2, lin_b2, bn_g2, bn_b2):
    pad = EPW - EPS16
    src2 = jnp.pad(edge_index[0].reshape(NSUB, EPS16), ((0, 0), (0, pad)))
    attr2 = jnp.pad(edge_attr.reshape(NSUB, EPS16), ((0, 0), (0, pad)))
    dst2 = jnp.pad(edge_index[1].reshape(NSUB, EPS16), ((0, 0), (0, pad)),
                   constant_values=N)
    gidx, dstp = _prep_call(src2, attr2, dst2)
    gidx3 = gidx.reshape(NSUB, CH, K)
    dst3 = dstp.reshape(NSUB * 2, HCH, K)
    batch3 = batch.reshape(NB, 1, BN)

    params = [
        (conv_w0, conv_b0, lin_w0, lin_b0, bn_g0, bn_b0),
        (conv_w1, conv_b1, lin_w1, lin_b1, bn_g1, bn_b1),
        (conv_w2, conv_b2, lin_w2, lin_b2, bn_g2, bn_b2),
    ]
    h = x
    pooled = []
    for l, (cw, cb, lw, lb, bg, bb) in enumerate(params):
        y0, y1 = _ymat_call(h, cw)
        a0, a1 = _sc_apply(y0, y1, gidx3, dst3)
        hr, stats, seg, cnt = _layer_call(h, a0, a1, lw, lb.reshape(1, DIM),
                                          cb, batch3)
        bg2, bb2 = bg.reshape(1, DIM), bb.reshape(1, DIM)
        if l < 2:
            h = _norm_call(hr, stats, bg2, bb2)
        pooled.append(_pool_call(seg, cnt, stats, bg2, bb2))
    return jnp.concatenate(pooled, axis=1)


# speedup vs baseline: 7.4573x; 1.0194x over previous
import functools

import jax
import jax.numpy as jnp
from jax import lax
from jax.experimental import pallas as pl
from jax.experimental.pallas import tpu as pltpu
from jax.experimental.pallas import tpu_sc as plsc

N = 10000
E = 160000
G = 64
DIM = 256
NE = 3
HALF = 128
NSUB = 16
EPS16 = E // NSUB
K = 128
EPW = 10240
CH = EPW // K
HCH = CH // 2
ACC = 10240
SRPS = ACC // NSUB
CSLAB = 200
BN = 1000
NB = N // BN
EPS = 1e-5


def _prep_body(src_ref, attr_ref, dst_ref, g_ref, dp_ref):
    g_ref[...] = attr_ref[...] * N + src_ref[...]
    dst = dst_ref[...]
    rowi = lax.broadcasted_iota(jnp.int32, (NSUB, EPW), 0)
    dp_ref[...] = jnp.where(dst < N, dst, N + rowi)


def _prep_call(src2, attr2, dst2):
    return pl.pallas_call(
        _prep_body,
        out_shape=[jax.ShapeDtypeStruct((NSUB, EPW), jnp.int32)] * 2,
    )(src2, attr2, dst2)


def _bn_scale_shift(stats_ref, g_ref, b_ref):
    mean = stats_ref[0:1, :] * (1.0 / N)
    var = stats_ref[1:2, :] * (1.0 / N) - mean * mean
    sc = lax.rsqrt(var + EPS) * g_ref[...]
    sh = b_ref[...] - mean * sc
    return sc, sh


def _ymat_body(h_ref, w_ref, y0_ref, y1_ref):
    w = w_ref[0]
    out = lax.dot_general(h_ref[...], w, (((1,), (1,)), ((), ())),
                          preferred_element_type=jnp.float32)
    y0_ref[...] = out[:, :HALF]
    y1_ref[...] = out[:, HALF:]


def _ymat_bn_body(h_ref, w_ref, stats_ref, g_ref, b_ref, y0_ref, y1_ref):
    w = w_ref[0]
    sc, sh = _bn_scale_shift(stats_ref, g_ref, b_ref)
    out = lax.dot_general(h_ref[...], w * sc, (((1,), (1,)), ((), ())),
                          preferred_element_type=jnp.float32)
    out = out + lax.dot_general(sh, w, (((1,), (1,)), ((), ())),
                                preferred_element_type=jnp.float32)
    y0_ref[...] = out[:, :HALF]
    y1_ref[...] = out[:, HALF:]


def _ymat_call(h, cw, bn=None):
    din = h.shape[1]
    yspec = [
        pl.BlockSpec((BN, HALF), lambda et, i: (et * NB + i, 0)),
        pl.BlockSpec((BN, HALF), lambda et, i: (et * NB + i, 0)),
    ]
    yshape = [jax.ShapeDtypeStruct((NE * N, HALF), jnp.float32)] * 2
    hw_specs = [
        pl.BlockSpec((BN, din), lambda et, i: (i, 0)),
        pl.BlockSpec((1, DIM, din), lambda et, i: (et, 0, 0)),
    ]
    if bn is None:
        return pl.pallas_call(
            _ymat_body, grid=(NE, NB), in_specs=hw_specs,
            out_specs=yspec, out_shape=yshape)(h, cw)
    stats, bg2, bb2 = bn
    return pl.pallas_call(
        _ymat_bn_body,
        grid=(NE, NB),
        in_specs=hw_specs + [
            pl.BlockSpec((8, din), lambda et, i: (0, 0)),
            pl.BlockSpec((1, din), lambda et, i: (0, 0)),
            pl.BlockSpec((1, din), lambda et, i: (0, 0)),
        ],
        out_specs=yspec, out_shape=yshape)(h, cw, stats, bg2, bb2)


def _layer_core(i, h, a0_ref, a1_ref, lw, bias, b3_ref,
                hr_ref, stats_ref, seg_ref, cnt_ref, pool_ref,
                og_ref, ob_ref):
    out = lax.dot_general(h, lw, (((1,), (1,)), ((), ())),
                          preferred_element_type=jnp.float32)
    out = out + bias
    out = out + jnp.concatenate([a0_ref[...], a1_ref[...]], axis=1)
    hr = jnp.maximum(out, 0.0)
    hr_ref[...] = hr

    @pl.when(i == 0)
    def _():
        stats_ref[...] = jnp.zeros_like(stats_ref)
        seg_ref[...] = jnp.zeros_like(seg_ref)
        cnt_ref[...] = jnp.zeros_like(cnt_ref)

    stats_ref[0:1, :] += jnp.sum(hr, axis=0)[None, :]
    stats_ref[1:2, :] += jnp.sum(hr * hr, axis=0)[None, :]
    b = b3_ref[0]
    oh = (lax.broadcasted_iota(jnp.int32, (G, BN), 0) == b).astype(jnp.float32)
    seg_ref[...] += lax.dot_general(oh, hr, (((1,), (0,)), ((), ())),
                                    preferred_element_type=jnp.float32)
    cnt_ref[...] += jnp.broadcast_to(jnp.sum(oh, axis=1)[:, None], (G, HALF))

    @pl.when(i == NB - 1)
    def _():
        sc, sh = _bn_scale_shift(stats_ref, og_ref, ob_ref)
        cnt = cnt_ref[:, 0:1]
        pooled = seg_ref[...] * sc + cnt * sh
        pool_ref[...] = pooled * lax.rsqrt(cnt)


def _layer_body(h_ref, a0_ref, a1_ref, lw_ref, lb_ref, cb_ref, b3_ref,
                og_ref, ob_ref,
                hr_ref, stats_ref, seg_ref, cnt_ref, pool_ref):
    i = pl.program_id(0)
    bias = (lb_ref[0, :] + cb_ref[0, :] + cb_ref[1, :] + cb_ref[2, :])[None, :]
    _layer_core(i, h_ref[...], a0_ref, a1_ref, lw_ref[...], bias, b3_ref,
                hr_ref, stats_ref, seg_ref, cnt_ref, pool_ref, og_ref, ob_ref)


def _layer_bn_body(h_ref, a0_ref, a1_ref, lw_ref, lb_ref, cb_ref, b3_ref,
                   og_ref, ob_ref, stats_ref_in, g_ref, b_ref,
                   hr_ref, stats_ref, seg_ref, cnt_ref, pool_ref):
    i = pl.program_id(0)
    lw = lw_ref[...]
    sc, sh = _bn_scale_shift(stats_ref_in, g_ref, b_ref)
    bias = (lb_ref[0, :] + cb_ref[0, :] + cb_ref[1, :] + cb_ref[2, :])[None, :]
    bias = bias + lax.dot_general(sh, lw, (((1,), (1,)), ((), ())),
                                  preferred_element_type=jnp.float32)
    _layer_core(i, h_ref[...], a0_ref, a1_ref, lw * sc, bias, b3_ref,
                hr_ref, stats_ref, seg_ref, cnt_ref, pool_ref, og_ref, ob_ref)


def _layer_call(h, a0, a1, lw, lb2, cb, batch3, og2, ob2, bn=None):
    din = h.shape[1]
    in_specs = [
        pl.BlockSpec((BN, din), lambda i: (i, 0)),
        pl.BlockSpec((BN, HALF), lambda i: (i, 0)),
        pl.BlockSpec((BN, HALF), lambda i: (i, 0)),
        pl.BlockSpec((DIM, din), lambda i: (0, 0)),
        pl.BlockSpec((1, DIM), lambda i: (0, 0)),
        pl.BlockSpec((NE, DIM), lambda i: (0, 0)),
        pl.BlockSpec((1, 1, BN), lambda i: (i, 0, 0)),
        pl.BlockSpec((1, DIM), lambda i: (0, 0)),
        pl.BlockSpec((1, DIM), lambda i: (0, 0)),
    ]
    out_specs = [
        pl.BlockSpec((BN, DIM), lambda i: (i, 0)),
        pl.BlockSpec((8, DIM), lambda i: (0, 0)),
        pl.BlockSpec((G, DIM), lambda i: (0, 0)),
        pl.BlockSpec((G, HALF), lambda i: (0, 0)),
        pl.BlockSpec((G, DIM), lambda i: (0, 0)),
    ]
    out_shape = [
        jax.ShapeDtypeStruct((N, DIM), jnp.float32),
        jax.ShapeDtypeStruct((8, DIM), jnp.float32),
        jax.ShapeDtypeStruct((G, DIM), jnp.float32),
        jax.ShapeDtypeStruct((G, HALF), jnp.float32),
        jax.ShapeDtypeStruct((G, DIM), jnp.float32),
    ]
    args = (h, a0, a1, lw, lb2, cb, batch3, og2, ob2)
    if bn is None:
        return pl.pallas_call(
            _layer_body, grid=(NB,), in_specs=in_specs,
            out_specs=out_specs, out_shape=out_shape)(*args)
    stats, bg2, bb2 = bn
    in_specs += [
        pl.BlockSpec((8, din), lambda i: (0, 0)),
        pl.BlockSpec((1, din), lambda i: (0, 0)),
        pl.BlockSpec((1, din), lambda i: (0, 0)),
    ]
    return pl.pallas_call(
        _layer_bn_body, grid=(NB,), in_specs=in_specs,
        out_specs=out_specs, out_shape=out_shape)(*args, stats, bg2, bb2)


@functools.lru_cache(maxsize=None)
def _sc_scatter_kernel():
    mesh = plsc.VectorSubcoreMesh(core_axis_name="c", subcore_axis_name="s")

    @functools.partial(
        pl.kernel,
        out_type=(jax.ShapeDtypeStruct((N, HALF), jnp.float32),
                  jax.ShapeDtypeStruct((N, HALF), jnp.float32)),
        mesh=mesh,
        scratch_types=[
            pltpu.VMEM((CH, K), jnp.int32),
            pltpu.VMEM((HCH, K), jnp.int32),
            pltpu.VMEM((K, HALF), jnp.float32),
            pltpu.VMEM((K, HALF), jnp.float32),
            pltpu.VMEM_SHARED((ACC, HALF), jnp.float32),
            pltpu.SemaphoreType.DMA,
            pltpu.SemaphoreType.DMA,
        ],
    )
    def _sc_scatter(y0_hbm, y1_hbm, gidx_hbm, dst_hbm,
                    agg0_hbm, agg1_hbm,
                    gidx_t, dst_t, rows0, rows1, acc, sem0, sem1):
        c = lax.axis_index("c")
        s = lax.axis_index("s")
        pltpu.sync_copy(gidx_hbm.at[s], gidx_t)

        @pl.loop(0, K)
        def _(r):
            @pl.loop(0, HALF, step=16)
            def _(f):
                rows0[r, pl.ds(f, 16)] = jnp.zeros((16,), jnp.float32)

        zbase = s * SRPS
        for k in range(SRPS // K):
            pltpu.sync_copy(rows0, acc.at[pl.ds(zbase + k * K, K)])
        plsc.subcore_barrier()

        def run_core(y_hbm, agg_hbm):
            bufs = (rows0, rows1)
            sems = (sem0, sem1)
            for half in range(2):
                pltpu.sync_copy(dst_hbm.at[2 * s + half], dst_t)
                for b in range(2):
                    pltpu.make_async_copy(
                        y_hbm.at[gidx_t.at[half * HCH + b]], bufs[b],
                        sems[b]).start()

                @pl.loop(0, HCH, step=2)
                def _(j):
                    for b in range(2):
                        jj = j + b
                        pltpu.make_async_copy(
                            y_hbm.at[gidx_t.at[half * HCH + jj]], bufs[b],
                            sems[b]).wait()
                        pltpu.sync_copy(bufs[b], acc.at[dst_t.at[jj]],
                                        add=True)

                        @pl.when(jj + 2 < HCH)
                        def _():
                            pltpu.make_async_copy(
                                y_hbm.at[gidx_t.at[half * HCH + jj + 2]],
                                bufs[b], sems[b]).start()

            plsc.subcore_barrier()

            for q in range(3):
                slab = s + 16 * q
                pltpu.sync_copy(
                    acc.at[pl.ds(slab * CSLAB, CSLAB)],
                    agg_hbm.at[pl.ds(slab * CSLAB, CSLAB)])

            @pl.when(s < 2)
            def _():
                slab = s + 48
                pltpu.sync_copy(
                    acc.at[pl.ds(slab * CSLAB, CSLAB)],
                    agg_hbm.at[pl.ds(slab * CSLAB, CSLAB)])

        @pl.when(c == 0)
        def _():
            run_core(y0_hbm, agg0_hbm)

        @pl.when(c == 1)
        def _():
            run_core(y1_hbm, agg1_hbm)

    return _sc_scatter


def _sc_apply(y0, y1, gidx3, dst3):
    return _sc_scatter_kernel()(y0, y1, gidx3, dst3)


def kernel(x, edge_index, edge_attr, batch,
           conv_w0, conv_b0, lin_w0, lin_b0, bn_g0, bn_b0,
           conv_w1, conv_b1, lin_w1, lin_b1, bn_g1, bn_b1,
           conv_w2, conv_b2, lin_w2, lin_b2, bn_g2, bn_b2):
    pad = EPW - EPS16
    src2 = jnp.pad(edge_index[0].reshape(NSUB, EPS16), ((0, 0), (0, pad)))
    attr2 = jnp.pad(edge_attr.reshape(NSUB, EPS16), ((0, 0), (0, pad)))
    dst2 = jnp.pad(edge_index[1].reshape(NSUB, EPS16), ((0, 0), (0, pad)),
                   constant_values=N)
    gidx, dstp = _prep_call(src2, attr2, dst2)
    gidx3 = gidx.reshape(NSUB, CH, K)
    dst3 = dstp.reshape(NSUB * 2, HCH, K)
    batch3 = batch.reshape(NB, 1, BN)

    params = [
        (conv_w0, conv_b0, lin_w0, lin_b0, bn_g0, bn_b0),
        (conv_w1, conv_b1, lin_w1, lin_b1, bn_g1, bn_b1),
        (conv_w2, conv_b2, lin_w2, lin_b2, bn_g2, bn_b2),
    ]
    h = x
    bn = None
    pooled = []
    for cw, cb, lw, lb, bg, bb in params:
        bg2, bb2 = bg.reshape(1, DIM), bb.reshape(1, DIM)
        y0, y1 = _ymat_call(h, cw, bn=bn)
        a0, a1 = _sc_apply(y0, y1, gidx3, dst3)
        h, stats, seg, cnt, pool = _layer_call(
            h, a0, a1, lw, lb.reshape(1, DIM), cb, batch3, bg2, bb2, bn=bn)
        bn = (stats, bg2, bb2)
        pooled.append(pool)
    return jnp.concatenate(pooled, axis=1)
